# baseline (speedup 1.0000x reference)
import jax
import jax.numpy as jnp
from jax import lax
from jax.experimental import pallas as pl
from jax.experimental.pallas import tpu as pltpu

B, H, L_Q, U, L_K = 4, 16, 4096, 128, 4096

ROWS_PER_BLOCK = 512
N_ROWS = B * H * U
N_BLOCKS = N_ROWS // ROWS_PER_BLOCK


def _mask_kernel(idx_ref, out_ref, scratch, sems):
    i = pl.program_id(0)
    slot = lax.rem(i, 2)

    def copy_for(block, use_slot):
        return pltpu.make_async_copy(
            scratch.at[use_slot],
            out_ref.at[pl.ds(block * ROWS_PER_BLOCK, ROWS_PER_BLOCK), :],
            sems.at[use_slot],
        )

    @pl.when(i >= 2)
    def _():
        copy_for(i - 2, slot).wait()

    col = jax.lax.broadcasted_iota(jnp.int16, (ROWS_PER_BLOCK, L_K), 1)
    scratch[slot] = (col > idx_ref[...]).astype(jnp.int8)
    copy_for(i, slot).start()

    @pl.when(i == N_BLOCKS - 1)
    def _():
        copy_for(i - 1, 1 - slot).wait()
        copy_for(i, slot).wait()


def kernel(index, scores):
    del scores
    idx = index.reshape(N_ROWS, 1).astype(jnp.int16)
    out = pl.pallas_call(
        _mask_kernel,
        grid=(N_BLOCKS,),
        in_specs=[pl.BlockSpec((ROWS_PER_BLOCK, 1), lambda i: (i, 0))],
        out_specs=pl.BlockSpec(memory_space=pltpu.HBM),
        out_shape=jax.ShapeDtypeStruct((N_ROWS, L_K), jnp.int8),
        scratch_shapes=[
            pltpu.VMEM((2, ROWS_PER_BLOCK, L_K), jnp.int8),
            pltpu.SemaphoreType.DMA((2,)),
        ],
    )(idx)
    return (out != 0).reshape(B, H, U, L_K)


# speedup vs baseline: 1.0365x; 1.0365x over previous
import jax
import jax.numpy as jnp
from jax.experimental import pallas as pl

B, H, L_Q, U, L_K = 4, 16, 4096, 128, 4096

ROWS_PER_BLOCK = 2048
N_ROWS = B * H * U
N_BLOCKS = N_ROWS // ROWS_PER_BLOCK


def _mask_kernel(idx_ref, out_ref):
    col = jax.lax.broadcasted_iota(jnp.int16, out_ref.shape, 1)
    out_ref[...] = (col > idx_ref[...]).astype(jnp.int8)


def kernel(index, scores):
    del scores
    idx = index.reshape(N_ROWS, 1).astype(jnp.int16)
    out = pl.pallas_call(
        _mask_kernel,
        grid=(N_BLOCKS,),
        in_specs=[pl.BlockSpec((ROWS_PER_BLOCK, 1), lambda i: (i, 0))],
        out_specs=pl.BlockSpec((ROWS_PER_BLOCK, L_K), lambda i: (i, 0)),
        out_shape=jax.ShapeDtypeStruct((N_ROWS, L_K), jnp.int8),
    )(idx)
    return (out != 0).reshape(B, H, U, L_K)
